# baseline (speedup 1.0000x reference)
import jax
import jax.numpy as jnp
from jax.experimental import pallas as pl


def _chamfer_kernel(x1_ref, x2_ref, dist1_ref, dist2_ref):
    m_idx = pl.program_id(1)

    x1 = x1_ref[0]
    x2 = x2_ref[0]

    sq1 = jnp.sum(x1 * x1, axis=0)
    sq2 = jnp.sum(x2 * x2, axis=0)

    cross = jax.lax.dot_general(
        x1, x2, (((0,), (0,)), ((), ())), preferred_element_type=jnp.float32
    )

    d = sq1[:, None] + sq2[None, :] - 2.0 * cross

    tile_min1 = jnp.min(d, axis=1)
    dist2_ref[0, 0] = jnp.min(d, axis=0)

    @pl.when(m_idx == 0)
    def _init():
        dist1_ref[0, 0] = tile_min1

    @pl.when(m_idx != 0)
    def _acc():
        dist1_ref[0, 0] = jnp.minimum(dist1_ref[0, 0], tile_min1)


def kernel(input1, input2):
    B, N, _ = input1.shape
    M = input2.shape[1]
    TM = 1024

    x1t = jnp.transpose(input1, (0, 2, 1))
    x2t = jnp.transpose(input2, (0, 2, 1))

    dist1, dist2 = pl.pallas_call(
        _chamfer_kernel,
        grid=(B, M // TM),
        in_specs=[
            pl.BlockSpec((1, 3, N), lambda b, m: (b, 0, 0)),
            pl.BlockSpec((1, 3, TM), lambda b, m: (b, 0, m)),
        ],
        out_specs=[
            pl.BlockSpec((1, 1, N), lambda b, m: (b, 0, 0)),
            pl.BlockSpec((1, 1, TM), lambda b, m: (b, 0, m)),
        ],
        out_shape=[
            jax.ShapeDtypeStruct((B, 1, N), jnp.float32),
            jax.ShapeDtypeStruct((B, 1, M), jnp.float32),
        ],
    )(x1t, x2t)

    return dist1[:, 0, :], dist2[:, 0, :]
